# baseline (speedup 1.0000x reference)
import jax
import jax.numpy as jnp
from jax import lax
from jax.experimental import pallas as pl
from jax.experimental.pallas import tpu as pltpu
from jax.experimental.pallas import tpu_sc as plsc

_N = 16777216
_NBINS = 256
_NW = 32
_PER_W = _N // _NW
_CHUNK = 16384
_NCHUNK = _PER_W // _CHUNK
_L = 16
_SCALE = 2.0 / (_NBINS - 1)


_STR = 17
_TABX = _NBINS * _STR


def _hist_equal_body(y_hbm, bins_hbm, incs_hbm, out_hbm, bins_v, rincs_v,
                     binsx_v, rincsx_v, ccx_v, yb, ob, yb2, ob2,
                     in_sem0, in_sem1, out_sem0, out_sem1):
    wid = lax.axis_index("s") * 2 + lax.axis_index("c")
    base = wid * _PER_W

    pltpu.sync_copy(bins_hbm, bins_v)
    pltpu.sync_copy(incs_hbm, rincs_v)

    lane = jnp.arange(_L, dtype=jnp.int32)

    @plsc.parallel_loop(0, _NBINS, step=_L)
    def _(o):
        rincs_v[pl.ds(o, _L)] = jnp.float32(_SCALE) / rincs_v[pl.ds(o, _L)]

    @plsc.parallel_loop(0, _NBINS, step=_L)
    def _(o):
        bv = bins_v[pl.ds(o, _L)]
        rv = rincs_v[pl.ds(o, _L)]
        cc = (o + lane).astype(jnp.float32) * jnp.float32(_SCALE) - bv * rv - 1.0
        addr0 = (o + lane) * _STR
        for l in range(_L):
            plsc.store_scatter(binsx_v, [addr0 + l], bv)
            plsc.store_scatter(rincsx_v, [addr0 + l], rv)
            plsc.store_scatter(ccx_v, [addr0 + l], cc)

    def in_copy(g, buf, sem):
        off = base + jnp.minimum(g, _NCHUNK - 1) * _CHUNK
        return pltpu.make_async_copy(y_hbm.at[pl.ds(off, _CHUNK)], buf, sem)

    def out_copy(g, buf, sem):
        off = base + g * _CHUNK
        return pltpu.make_async_copy(buf, out_hbm.at[pl.ds(off, _CHUNK)], sem)

    def compute(src, dst):
        t1 = lane + 128 * _STR
        b128 = plsc.load_gather(binsx_v, [t1])
        b64 = plsc.load_gather(binsx_v, [lane + 64 * _STR])
        b192 = plsc.load_gather(binsx_v, [lane + 192 * _STR])
        b_lo = plsc.load_gather(binsx_v, [lane])
        b255 = plsc.load_gather(binsx_v, [lane + 255 * _STR])
        r255 = plsc.load_gather(rincsx_v, [lane + 255 * _STR])
        b_hi = b255 + jnp.float32(_SCALE) / r255

        @plsc.parallel_loop(0, _CHUNK, step=_L, unroll=8)
        def _(o):
            v = jnp.minimum(jnp.maximum(src[pl.ds(o, _L)], b_lo), b_hi)
            m1 = b128 <= v
            k = jnp.where(m1, t1, lane)
            bt2 = jnp.where(m1, b192, b64)
            t = k + 64 * _STR
            k = jnp.where(bt2 <= v, t, k)
            for s in (32, 16, 8, 4, 2, 1):
                t = k + s * _STR
                bt = plsc.load_gather(binsx_v, [t])
                k = jnp.where(bt <= v, t, k)
            ri = plsc.load_gather(rincsx_v, [k])
            cc = plsc.load_gather(ccx_v, [k])
            dst[pl.ds(o, _L)] = v * ri + cc

    in_copy(0, yb, in_sem0).start()
    in_copy(1, yb2, in_sem1).start()

    def pair_body(p, c):
        g0 = 2 * p
        g1 = g0 + 1
        in_copy(g0, yb, in_sem0).wait()

        @pl.when(p > 0)
        def _():
            out_copy(g0, ob, out_sem0).wait()

        compute(yb, ob)
        out_copy(g0, ob, out_sem0).start()
        in_copy(g0 + 2, yb, in_sem0).start()

        in_copy(g1, yb2, in_sem1).wait()

        @pl.when(p > 0)
        def _():
            out_copy(g1, ob2, out_sem1).wait()

        compute(yb2, ob2)
        out_copy(g1, ob2, out_sem1).start()
        in_copy(g1 + 2, yb2, in_sem1).start()
        return c

    lax.fori_loop(0, _NCHUNK // 2, pair_body, 0)
    out_copy(_NCHUNK - 2, ob, out_sem0).wait()
    out_copy(_NCHUNK - 1, ob2, out_sem1).wait()
    in_copy(_NCHUNK - 1, yb, in_sem0).wait()
    in_copy(_NCHUNK - 1, yb2, in_sem1).wait()


def kernel(y, bins, incs):
    mesh = plsc.VectorSubcoreMesh(core_axis_name="c", subcore_axis_name="s")
    f = pl.kernel(
        _hist_equal_body,
        out_type=jax.ShapeDtypeStruct((_N,), jnp.float32),
        mesh=mesh,
        scratch_types=[
            pltpu.VMEM((_NBINS,), jnp.float32),
            pltpu.VMEM((_NBINS,), jnp.float32),
            pltpu.VMEM((_TABX,), jnp.float32),
            pltpu.VMEM((_TABX,), jnp.float32),
            pltpu.VMEM((_TABX,), jnp.float32),
            pltpu.VMEM((_CHUNK,), jnp.float32),
            pltpu.VMEM((_CHUNK,), jnp.float32),
            pltpu.VMEM((_CHUNK,), jnp.float32),
            pltpu.VMEM((_CHUNK,), jnp.float32),
            pltpu.SemaphoreType.DMA,
            pltpu.SemaphoreType.DMA,
            pltpu.SemaphoreType.DMA,
            pltpu.SemaphoreType.DMA,
        ],
        compiler_params=pltpu.CompilerParams(needs_layout_passes=False),
    )
    return f(y, bins, incs)


# speedup vs baseline: 1.2619x; 1.2619x over previous
import jax
import jax.numpy as jnp
from jax import lax
from jax.experimental import pallas as pl
from jax.experimental.pallas import tpu as pltpu
from jax.experimental.pallas import tpu_sc as plsc

_N = 16777216
_NBINS = 256
_NW = 32
_PER_W = _N // _NW
_CHUNK = 16384
_NCHUNK = _PER_W // _CHUNK
_L = 16
_SCALE = 2.0 / (_NBINS - 1)


_STR = 17
_TABX = _NBINS * _STR


def _hist_equal_body(y_hbm, bins_hbm, incs_hbm, out_hbm, bins_v, rincs_v,
                     binsx_v, rincsx_v, yb, ob, yb2, ob2,
                     in_sem0, in_sem1, out_sem0, out_sem1):
    wid = lax.axis_index("s") * 2 + lax.axis_index("c")
    base = wid * _PER_W

    pltpu.sync_copy(bins_hbm, bins_v)
    pltpu.sync_copy(incs_hbm, rincs_v)

    lane = jnp.arange(_L, dtype=jnp.int32)

    @plsc.parallel_loop(0, _NBINS, step=_L)
    def _(o):
        rincs_v[pl.ds(o, _L)] = jnp.float32(_SCALE) / rincs_v[pl.ds(o, _L)]

    @plsc.parallel_loop(0, _NBINS, step=_L)
    def _(o):
        bv = bins_v[pl.ds(o, _L)]
        rv = rincs_v[pl.ds(o, _L)]
        addr0 = (o + lane) * _STR
        for l in range(_L):
            plsc.store_scatter(binsx_v, [addr0 + l], bv)
            plsc.store_scatter(rincsx_v, [addr0 + l], rv)

    def in_copy(g, buf, sem):
        off = base + jnp.minimum(g, _NCHUNK - 1) * _CHUNK
        return pltpu.make_async_copy(y_hbm.at[pl.ds(off, _CHUNK)], buf, sem)

    def out_copy(g, buf, sem):
        off = base + g * _CHUNK
        return pltpu.make_async_copy(buf, out_hbm.at[pl.ds(off, _CHUNK)], sem)

    def compute(src, dst):
        t1 = lane + 128 * _STR
        b128 = plsc.load_gather(binsx_v, [t1])
        b64 = plsc.load_gather(binsx_v, [lane + 64 * _STR])
        b192 = plsc.load_gather(binsx_v, [lane + 192 * _STR])

        @plsc.parallel_loop(0, _CHUNK, step=_L, unroll=4)
        def _(o):
            v = src[pl.ds(o, _L)]
            m1 = b128 <= v
            k = jnp.where(m1, t1, lane)
            bt2 = jnp.where(m1, b192, b64)
            t = k + 64 * _STR
            k = jnp.where(bt2 <= v, t, k)
            for s in (32, 16, 8, 4, 2, 1):
                t = k + s * _STR
                bt = plsc.load_gather(binsx_v, [t])
                k = jnp.where(bt <= v, t, k)
            bv = plsc.load_gather(binsx_v, [k])
            ri = plsc.load_gather(rincsx_v, [k])
            d = jnp.minimum(jnp.maximum((v - bv) * ri, 0.0), jnp.float32(_SCALE))
            kf = (k - lane).astype(jnp.float32) * jnp.float32(_SCALE / _STR)
            dst[pl.ds(o, _L)] = kf + d - 1.0

    in_copy(0, yb, in_sem0).start()
    in_copy(1, yb2, in_sem1).start()

    def pair_body(p, c):
        g0 = 2 * p
        g1 = g0 + 1
        in_copy(g0, yb, in_sem0).wait()

        @pl.when(p > 0)
        def _():
            out_copy(g0, ob, out_sem0).wait()

        compute(yb, ob)
        out_copy(g0, ob, out_sem0).start()
        in_copy(g0 + 2, yb, in_sem0).start()

        in_copy(g1, yb2, in_sem1).wait()

        @pl.when(p > 0)
        def _():
            out_copy(g1, ob2, out_sem1).wait()

        compute(yb2, ob2)
        out_copy(g1, ob2, out_sem1).start()
        in_copy(g1 + 2, yb2, in_sem1).start()
        return c

    lax.fori_loop(0, _NCHUNK // 2, pair_body, 0)
    out_copy(_NCHUNK - 2, ob, out_sem0).wait()
    out_copy(_NCHUNK - 1, ob2, out_sem1).wait()
    in_copy(_NCHUNK - 1, yb, in_sem0).wait()
    in_copy(_NCHUNK - 1, yb2, in_sem1).wait()


def kernel(y, bins, incs):
    mesh = plsc.VectorSubcoreMesh(core_axis_name="c", subcore_axis_name="s")
    f = pl.kernel(
        _hist_equal_body,
        out_type=jax.ShapeDtypeStruct((_N,), jnp.float32),
        mesh=mesh,
        scratch_types=[
            pltpu.VMEM((_NBINS,), jnp.float32),
            pltpu.VMEM((_NBINS,), jnp.float32),
            pltpu.VMEM((_TABX,), jnp.float32),
            pltpu.VMEM((_TABX,), jnp.float32),
            pltpu.VMEM((_CHUNK,), jnp.float32),
            pltpu.VMEM((_CHUNK,), jnp.float32),
            pltpu.VMEM((_CHUNK,), jnp.float32),
            pltpu.VMEM((_CHUNK,), jnp.float32),
            pltpu.SemaphoreType.DMA,
            pltpu.SemaphoreType.DMA,
            pltpu.SemaphoreType.DMA,
            pltpu.SemaphoreType.DMA,
        ],
        compiler_params=pltpu.CompilerParams(needs_layout_passes=False),
    )
    return f(y, bins, incs)


# speedup vs baseline: 1.2911x; 1.0231x over previous
import jax
import jax.numpy as jnp
from jax import lax
from jax.experimental import pallas as pl
from jax.experimental.pallas import tpu as pltpu
from jax.experimental.pallas import tpu_sc as plsc

_N = 16777216
_NBINS = 256
_NW = 32
_PER_W = _N // _NW
_CHUNK = 16384
_NCHUNK = _PER_W // _CHUNK
_L = 16
_SCALE = 2.0 / (_NBINS - 1)


_STR = 17
_TABX = _NBINS * _STR


def _hist_equal_body(y_hbm, bins_hbm, incs_hbm, out_hbm, bins_v, rincs_v,
                     binsx_v, rincsx_v, yb, ob, yb2, ob2,
                     in_sem0, in_sem1, out_sem0, out_sem1):
    wid = lax.axis_index("s") * 2 + lax.axis_index("c")
    base = wid * _PER_W

    pltpu.sync_copy(bins_hbm, bins_v)
    pltpu.sync_copy(incs_hbm, rincs_v)

    lane = jnp.arange(_L, dtype=jnp.int32)

    @plsc.parallel_loop(0, _NBINS, step=_L)
    def _(o):
        rincs_v[pl.ds(o, _L)] = jnp.float32(_SCALE) / rincs_v[pl.ds(o, _L)]

    @plsc.parallel_loop(0, _NBINS, step=_L)
    def _(o):
        bv = bins_v[pl.ds(o, _L)]
        rv = rincs_v[pl.ds(o, _L)]
        addr0 = (o + lane) * _STR
        for l in range(_L):
            plsc.store_scatter(binsx_v, [addr0 + l], bv)
            plsc.store_scatter(rincsx_v, [addr0 + l], rv)

    def in_copy(g, buf, sem):
        off = base + jnp.minimum(g, _NCHUNK - 1) * _CHUNK
        return pltpu.make_async_copy(y_hbm.at[pl.ds(off, _CHUNK)], buf, sem)

    def out_copy(g, buf, sem):
        off = base + g * _CHUNK
        return pltpu.make_async_copy(buf, out_hbm.at[pl.ds(off, _CHUNK)], sem)

    def compute(src, dst):
        t1 = lane + 128 * _STR
        b128 = plsc.load_gather(binsx_v, [t1])
        b64 = plsc.load_gather(binsx_v, [lane + 64 * _STR])
        b192 = plsc.load_gather(binsx_v, [lane + 192 * _STR])

        @plsc.parallel_loop(0, _CHUNK, step=_L, unroll=2)
        def _(o):
            v = src[pl.ds(o, _L)]
            m1 = b128 <= v
            k = jnp.where(m1, t1, lane)
            bt2 = jnp.where(m1, b192, b64)
            t = k + 64 * _STR
            k = jnp.where(bt2 <= v, t, k)
            for s in (32, 16, 8, 4, 2, 1):
                t = k + s * _STR
                bt = plsc.load_gather(binsx_v, [t])
                k = jnp.where(bt <= v, t, k)
            bv = plsc.load_gather(binsx_v, [k])
            ri = plsc.load_gather(rincsx_v, [k])
            d = jnp.minimum(jnp.maximum((v - bv) * ri, 0.0), jnp.float32(_SCALE))
            kf = (k - lane).astype(jnp.float32) * jnp.float32(_SCALE / _STR)
            dst[pl.ds(o, _L)] = kf + d - 1.0

    in_copy(0, yb, in_sem0).start()
    in_copy(1, yb2, in_sem1).start()

    def pair_body(p, c):
        g0 = 2 * p
        g1 = g0 + 1
        in_copy(g0, yb, in_sem0).wait()

        @pl.when(p > 0)
        def _():
            out_copy(g0, ob, out_sem0).wait()

        compute(yb, ob)
        out_copy(g0, ob, out_sem0).start()
        in_copy(g0 + 2, yb, in_sem0).start()

        in_copy(g1, yb2, in_sem1).wait()

        @pl.when(p > 0)
        def _():
            out_copy(g1, ob2, out_sem1).wait()

        compute(yb2, ob2)
        out_copy(g1, ob2, out_sem1).start()
        in_copy(g1 + 2, yb2, in_sem1).start()
        return c

    lax.fori_loop(0, _NCHUNK // 2, pair_body, 0)
    out_copy(_NCHUNK - 2, ob, out_sem0).wait()
    out_copy(_NCHUNK - 1, ob2, out_sem1).wait()
    in_copy(_NCHUNK - 1, yb, in_sem0).wait()
    in_copy(_NCHUNK - 1, yb2, in_sem1).wait()


def kernel(y, bins, incs):
    mesh = plsc.VectorSubcoreMesh(core_axis_name="c", subcore_axis_name="s")
    f = pl.kernel(
        _hist_equal_body,
        out_type=jax.ShapeDtypeStruct((_N,), jnp.float32),
        mesh=mesh,
        scratch_types=[
            pltpu.VMEM((_NBINS,), jnp.float32),
            pltpu.VMEM((_NBINS,), jnp.float32),
            pltpu.VMEM((_TABX,), jnp.float32),
            pltpu.VMEM((_TABX,), jnp.float32),
            pltpu.VMEM((_CHUNK,), jnp.float32),
            pltpu.VMEM((_CHUNK,), jnp.float32),
            pltpu.VMEM((_CHUNK,), jnp.float32),
            pltpu.VMEM((_CHUNK,), jnp.float32),
            pltpu.SemaphoreType.DMA,
            pltpu.SemaphoreType.DMA,
            pltpu.SemaphoreType.DMA,
            pltpu.SemaphoreType.DMA,
        ],
        compiler_params=pltpu.CompilerParams(needs_layout_passes=False),
    )
    return f(y, bins, incs)
